# baseline (speedup 1.0000x reference)
import functools

import jax
import jax.numpy as jnp
from jax import lax
from jax.experimental import pallas as pl
from jax.experimental.pallas import tpu as pltpu
from jax.experimental.pallas import tpu_sc as plsc

N = 10000
D = 128
DH = D // 2
E = 320000
NC = 2
NS = 16
B = 128
ET = E // NS
K = 160
ETP = K * B
NBUF = 4
NP = 10240
RPT = NP // NS

_MESH = plsc.VectorSubcoreMesh(
    core_axis_name="c", subcore_axis_name="s", num_cores=NC, num_subcores=NS
)


@functools.partial(
    pl.kernel,
    out_type=jax.ShapeDtypeStruct((NC, NP, DH), jnp.float32),
    mesh=_MESH,
    scratch_types=[
        pltpu.VMEM((K, B), jnp.int32),
        pltpu.VMEM((K, B), jnp.int32),
        [pltpu.VMEM((B, DH), jnp.float32)] * NBUF,
        pltpu.VMEM_SHARED((NP, DH), jnp.float32),
        [pltpu.SemaphoreType.DMA] * NBUF,
    ],
    compiler_params=pltpu.CompilerParams(use_tc_tiling_on_sc=False),
)
def _sc_aggregate(xr_hbm, col_hbm, row_hbm, zero_hbm, out_hbm,
                  colv, rowv, rbufs, acc, gsems):
    cid = lax.axis_index("c")
    sid = lax.axis_index("s")

    pltpu.sync_copy(col_hbm.at[cid * NS + sid], colv)
    pltpu.sync_copy(row_hbm.at[sid], rowv)
    pltpu.sync_copy(zero_hbm.at[pl.ds(sid * RPT, RPT)],
                    acc.at[pl.ds(sid * RPT, RPT)])
    plsc.subcore_barrier()

    for b in range(NBUF):
        pltpu.async_copy(xr_hbm.at[colv.at[b]], rbufs[b], gsems[b])

    def steady(jj, carry):
        for b in range(NBUF):
            j = NBUF * jj + b
            pltpu.make_async_copy(xr_hbm.at[colv.at[j]], rbufs[b],
                                  gsems[b]).wait()
            pltpu.sync_copy(rbufs[b], acc.at[rowv.at[j]], add=True)
            pltpu.async_copy(xr_hbm.at[colv.at[j + NBUF]], rbufs[b],
                             gsems[b])
        return carry

    lax.fori_loop(0, K // NBUF - 1, steady, 0)

    for b in range(NBUF):
        j = K - NBUF + b
        pltpu.make_async_copy(xr_hbm.at[colv.at[j]], rbufs[b],
                              gsems[b]).wait()
        pltpu.sync_copy(rbufs[b], acc.at[rowv.at[j]], add=True)
    plsc.subcore_barrier()

    pltpu.sync_copy(acc.at[pl.ds(sid * RPT, RPT)],
                    out_hbm.at[cid, pl.ds(sid * RPT, RPT)])


def _mm_body(p_ref, w_ref, o_ref):
    s = jnp.concatenate([p_ref[0], p_ref[1]], axis=1)
    o_ref[...] = jnp.maximum(
        jnp.dot(s, w_ref[...], preferred_element_type=jnp.float32), 0.0)


_BM = 1000


def _tc_combine_matmul(p, w):
    return pl.pallas_call(
        _mm_body,
        grid=(N // _BM,),
        in_specs=[
            pl.BlockSpec((NC, _BM, DH), lambda i: (0, i, 0)),
            pl.BlockSpec((D, D), lambda i: (0, 0)),
        ],
        out_specs=pl.BlockSpec((_BM, D), lambda i: (i, 0)),
        out_shape=jax.ShapeDtypeStruct((N, D), jnp.float32),
    )(p, w)


def _prep_edges(edge_index):
    npad = ETP - ET
    pad_col = jnp.zeros((NS, npad), jnp.int32)
    pad_row = jnp.broadcast_to(
        N + (jnp.arange(npad, dtype=jnp.int32) % (NP - N)), (NS, npad))
    col = jnp.concatenate([edge_index[1].reshape(NS, ET), pad_col], axis=1)
    row = jnp.concatenate([edge_index[0].reshape(NS, ET), pad_row], axis=1)
    colx = jnp.stack([2 * col, 2 * col + 1]).reshape(NC * NS, K, B)
    return colx, row.reshape(NS, K, B)


def kernel(x, edge_index0, edge_index1, W0, W1):
    col0, row0 = _prep_edges(edge_index0)
    col1, row1 = _prep_edges(edge_index1)
    zero = jnp.zeros((NP, DH), jnp.float32)

    p0 = _sc_aggregate(x.reshape(2 * N, DH), col0, row0, zero)
    h1 = _tc_combine_matmul(p0, W0)
    p1 = _sc_aggregate(h1.reshape(2 * N, DH), col1, row1, zero)
    return _tc_combine_matmul(p1, W1)
